# baseline (speedup 1.0000x reference)
import numpy as np

import jax
import jax.numpy as jnp
from jax import lax
from jax.experimental import pallas as pl
from jax.experimental.pallas import tpu as pltpu

_DT = jnp.bfloat16
_BB = 8
_C1, _C2, _C3, _C4, _C5 = 8, 16, 32, 32, 32

_CONV_CFG = [
    ("conv1_1", 3, _C1), ("conv1_2", _C1, _C1),
    ("conv2_1", _C1, _C2), ("conv2_2", _C2, _C2),
    ("conv3_1", _C2, _C3), ("conv3_2", _C3, _C3), ("conv3_3", _C3, _C3),
    ("conv4_1", _C3, _C4), ("conv4_2", _C4, _C4), ("conv4_3", _C4, _C4),
    ("conv5_1", _C4, _C5),
]

_PLAN = [
    ("conv1_1", "x",       "relu1_1", 64, 3,   _C1, None),
    ("conv1_2", "relu1_1", "relu1_2", 64, _C1, _C1, "p1"),
    ("conv2_1", "p1",      "relu2_1", 32, _C1, _C2, None),
    ("conv2_2", "relu2_1", "relu2_2", 32, _C2, _C2, "p2"),
    ("conv3_1", "p2",      "relu3_1", 16, _C2, _C3, None),
    ("conv3_2", "relu3_1", "relu3_2", 16, _C3, _C3, None),
    ("conv3_3", "relu3_2", "relu3_3", 16, _C3, _C3, "max_3"),
    ("conv4_1", "max_3",   "relu4_1", 8,  _C3, _C4, None),
    ("conv4_2", "relu4_1", "relu4_2", 8,  _C4, _C4, None),
    ("conv4_3", "relu4_2", "relu4_3", 8,  _C4, _C4, "p3"),
    ("conv5_1", "p3",      "relu5_1", 4,  _C4, _C5, "p4"),
    ("conv5_1", "p4",      "relu5_2", 2,  _C5, _C5, "p5"),
    ("conv5_1", "p5",      "relu5_3", 1,  _C5, _C5, None),
]

_OUTPUTS = [
    ("relu1_1", 64, _C1), ("relu1_2", 64, _C1),
    ("relu2_1", 32, _C2), ("relu2_2", 32, _C2),
    ("relu3_1", 16, _C3), ("relu3_2", 16, _C3), ("relu3_3", 16, _C3),
    ("max_3", 8, _C3),
    ("relu4_1", 8, _C4), ("relu4_2", 8, _C4), ("relu4_3", 8, _C4),
    ("relu5_1", 4, _C5), ("relu5_2", 2, _C5), ("relu5_3", 1, _C5),
]

_SCRATCH_ONLY = [("p1", 32, _C1), ("p2", 16, _C2), ("p3", 4, _C4),
                 ("p4", 2, _C5), ("p5", 1, _C5)]

_BUFFERS = [
    ("relu1_1", 64, _C1), ("relu1_2", 64, _C1), ("p1", 32, _C1),
    ("relu2_1", 32, _C2), ("relu2_2", 32, _C2), ("p2", 16, _C2),
    ("relu3_1", 16, _C3), ("relu3_2", 16, _C3), ("relu3_3", 16, _C3),
    ("max_3", 8, _C3),
    ("relu4_1", 8, _C4), ("relu4_2", 8, _C4), ("relu4_3", 8, _C4),
    ("p3", 4, _C4),
    ("relu5_1", 4, _C5), ("p4", 2, _C5), ("relu5_2", 2, _C5),
    ("p5", 1, _C5), ("relu5_3", 1, _C5),
]

_POOL_SIZES = [64, 32, 16, 8, 4, 2]


def _r(s):
    return (s + 2) * (s + 2)


def _pool_selector(s):
    wp, so = s + 2, s // 2
    wpo = so + 2
    k = (s - 1) * wp + s
    sel = np.zeros((k, wpo * wpo), np.float32)
    for i in range(so):
        for j in range(so):
            sel[(2 * i + 1) * wp + (2 * j + 1), (i + 1) * wpo + (j + 1)] = 1.0
    return sel.astype(jnp.bfloat16)


_SELS = {s: _pool_selector(s) for s in _POOL_SIZES}


def _interior_mask(s):
    wp, r = s + 2, _r(s)
    big = _BB * r
    m0, m1 = wp + 1, big - wp - 1
    col = np.arange(m0, m1)
    k = col % r
    h, w = k // wp, k % wp
    keep = (h >= 1) & (h <= s) & (w >= 1) & (w <= s)
    return keep[None, :].astype(np.float32)


_MASK_SIZES = [64, 32, 16, 8, 4, 2, 1]
_MASKS = {s: _interior_mask(s) for s in _MASK_SIZES}


def _conv3x3_relu(src, dst, w_ref, b_ref, sx, s, cin, cout, keep):
    wp, r = s + 2, _r(s)
    big = _BB * r
    m0, m1 = wp + 1, big - wp - 1
    mlen = m1 - m0
    k9 = 9 * cin

    offs = [(dy - 1) * wp + (dx - 1) for dy in range(3) for dx in range(3)]
    per = next((p for p in range(1, 9) if (p * cin) % 16 == 0), 9)
    t = 0
    while t < 9:
        g = offs[t:t + per]
        blk = jnp.concatenate(
            [src[:, m0 + o:m1 + o] for o in g], axis=0) if len(g) > 1 \
            else src[:, m0 + g[0]:m1 + g[0]]
        rows = len(g) * cin
        if rows % 16:
            blk = jnp.concatenate(
                [blk, jnp.zeros((16 - rows % 16, mlen), _DT)], axis=0)
            rows += 16 - rows % 16
        sx[t * cin:t * cin + rows, 0:mlen] = blk
        t += per

    acc = None
    for k0 in range(0, k9, 256):
        kc = min(256, k9 - k0)
        d = jnp.dot(w_ref[:, k0:k0 + kc], sx[k0:k0 + kc, 0:mlen],
                    preferred_element_type=jnp.float32)
        acc = d if acc is None else acc + d
    acc = jnp.maximum(acc + b_ref[...], 0.0) * keep[...]

    dst[:, 0:m0] = jnp.zeros((cout, m0), _DT)
    dst[:, m0:m1] = acc.astype(_DT)
    dst[:, m1:big] = jnp.zeros((cout, big - m1), _DT)


def _maxpool2x2(src, dst, sel_ref, s, c):
    wp, r = s + 2, _r(s)
    big = _BB * r
    ro = _r(s // 2)
    k = (s - 1) * wp + s

    v = src[...]
    a = jnp.maximum(v[:, 0:big - 1], v[:, 1:big])
    m2 = jnp.maximum(a[:, 0:big - 1 - wp], a[:, wp:big - 1])
    seg = jnp.concatenate([m2[:, e * r:e * r + k] for e in range(_BB)],
                          axis=0)
    o = jnp.dot(seg, sel_ref[...], preferred_element_type=jnp.float32)
    for e in range(_BB):
        dst[:, e * ro:(e + 1) * ro] = o[e * c:(e + 1) * c, :].astype(_DT)


def _net_kernel(*refs):
    pos = 0
    x_ref = refs[pos]
    pos += 1
    w, b = {}, {}
    for name, _, _ in _CONV_CFG:
        w[name], b[name] = refs[pos], refs[pos + 1]
        pos += 2
    sels = {}
    for s in _POOL_SIZES:
        sels[s] = refs[pos]
        pos += 1
    masks = {}
    for s in _MASK_SIZES:
        masks[s] = refs[pos]
        pos += 1
    buf = {"x": x_ref}
    for name, _, _ in _OUTPUTS:
        buf[name] = refs[pos]
        pos += 1
    for name, _, _ in _SCRATCH_ONLY:
        buf[name] = refs[pos]
        pos += 1
    sxa, sxb = refs[pos], refs[pos + 1]

    for cname, src, dst, s, cin, cout, pooled in _PLAN:
        _conv3x3_relu(buf[src], buf[dst], w[cname], b[cname],
                      sxa if s == 64 else sxb, s, cin, cout, masks[s])
        if pooled is not None:
            _maxpool2x2(buf[dst], buf[pooled], sels[s], s, cout)


def _build_call(batch):
    rx = _r(64)
    grid = (batch // _BB,)
    in_specs = [pl.BlockSpec((None, 3, _BB * rx), lambda n: (n, 0, 0))]
    for _, cin, cout in _CONV_CFG:
        in_specs.append(pl.BlockSpec((cout, 9 * cin), lambda n: (0, 0)))
        in_specs.append(pl.BlockSpec((cout, 1), lambda n: (0, 0)))
    for s in _POOL_SIZES:
        in_specs.append(pl.BlockSpec(_SELS[s].shape, lambda n: (0, 0)))
    for s in _MASK_SIZES:
        in_specs.append(pl.BlockSpec(_MASKS[s].shape, lambda n: (0, 0)))
    out_specs = tuple(pl.BlockSpec((None, c, _BB * _r(s)), lambda n: (n, 0, 0))
                      for _, s, c in _OUTPUTS)
    out_shape = tuple(jax.ShapeDtypeStruct((batch // _BB, c, _BB * _r(s)), _DT)
                      for _, s, c in _OUTPUTS)
    scratch = [pltpu.VMEM((c, _BB * _r(s)), _DT) for _, s, c in _SCRATCH_ONLY]
    scratch.append(pltpu.VMEM((80, _BB * rx), _DT))
    scratch.append(pltpu.VMEM((288, _BB * _r(32)), _DT))
    return pl.pallas_call(
        _net_kernel,
        grid=grid,
        in_specs=in_specs,
        out_specs=out_specs,
        out_shape=out_shape,
        scratch_shapes=scratch,
        compiler_params=pltpu.CompilerParams(
            dimension_semantics=("parallel",)),
    )


def kernel(x, w_conv1_1, b_conv1_1, w_conv1_2, b_conv1_2,
           w_conv2_1, b_conv2_1, w_conv2_2, b_conv2_2,
           w_conv3_1, b_conv3_1, w_conv3_2, b_conv3_2, w_conv3_3, b_conv3_3,
           w_conv4_1, b_conv4_1, w_conv4_2, b_conv4_2, w_conv4_3, b_conv4_3,
           w_conv5_1, b_conv5_1):
    wmap = {
        "conv1_1": (w_conv1_1, b_conv1_1), "conv1_2": (w_conv1_2, b_conv1_2),
        "conv2_1": (w_conv2_1, b_conv2_1), "conv2_2": (w_conv2_2, b_conv2_2),
        "conv3_1": (w_conv3_1, b_conv3_1), "conv3_2": (w_conv3_2, b_conv3_2),
        "conv3_3": (w_conv3_3, b_conv3_3),
        "conv4_1": (w_conv4_1, b_conv4_1), "conv4_2": (w_conv4_2, b_conv4_2),
        "conv4_3": (w_conv4_3, b_conv4_3),
        "conv5_1": (w_conv5_1, b_conv5_1),
    }
    batch = x.shape[0]
    h = x.shape[2]

    xp = jnp.pad(x, ((0, 0), (0, 0), (1, 1), (1, 1)))
    xp = xp.reshape(batch // _BB, _BB, 3, _r(h)).astype(_DT)
    xp = xp.transpose(0, 2, 1, 3).reshape(batch // _BB, 3, _BB * _r(h))

    args = [xp]
    for name, cin, cout in _CONV_CFG:
        wm, bm = wmap[name]
        args.append(wm)
        args.append(bm)
    for s in _POOL_SIZES:
        args.append(_SELS[s])
    for s in _MASK_SIZES:
        args.append(_MASKS[s])

    outs = _build_call(batch)(*args)

    feats = {}
    for (name, s, c), arr in zip(_OUTPUTS, outs):
        wp = s + 2
        a = arr.reshape(batch // _BB, c, _BB, wp, wp)
        a = a.transpose(0, 2, 1, 3, 4).reshape(batch, c, wp, wp)
        feats[name] = a[:, :, 1:-1, 1:-1].astype(jnp.float32)
    return feats


# speedup vs baseline: 1.3718x; 1.3718x over previous
import numpy as np

import jax
import jax.numpy as jnp
from jax import lax
from jax.experimental import pallas as pl
from jax.experimental.pallas import tpu as pltpu

_DT = jnp.bfloat16
_BB = 8
_C1, _C2, _C3, _C4, _C5 = 8, 16, 32, 32, 32

_CONV_CFG = [
    ("conv1_1", 3, _C1), ("conv1_2", _C1, _C1),
    ("conv2_1", _C1, _C2), ("conv2_2", _C2, _C2),
    ("conv3_1", _C2, _C3), ("conv3_2", _C3, _C3), ("conv3_3", _C3, _C3),
    ("conv4_1", _C3, _C4), ("conv4_2", _C4, _C4), ("conv4_3", _C4, _C4),
    ("conv5_1", _C4, _C5),
]

_PLAN = [
    ("conv1_1", "x",       "relu1_1", 64, 3,   _C1, None),
    ("conv1_2", "relu1_1", "relu1_2", 64, _C1, _C1, "p1"),
    ("conv2_1", "p1",      "relu2_1", 32, _C1, _C2, None),
    ("conv2_2", "relu2_1", "relu2_2", 32, _C2, _C2, "p2"),
    ("conv3_1", "p2",      "relu3_1", 16, _C2, _C3, None),
    ("conv3_2", "relu3_1", "relu3_2", 16, _C3, _C3, None),
    ("conv3_3", "relu3_2", "relu3_3", 16, _C3, _C3, "max_3"),
    ("conv4_1", "max_3",   "relu4_1", 8,  _C3, _C4, None),
    ("conv4_2", "relu4_1", "relu4_2", 8,  _C4, _C4, None),
    ("conv4_3", "relu4_2", "relu4_3", 8,  _C4, _C4, "p3"),
    ("conv5_1", "p3",      "relu5_1", 4,  _C4, _C5, "p4"),
    ("conv5_1", "p4",      "relu5_2", 2,  _C5, _C5, "p5"),
    ("conv5_1", "p5",      "relu5_3", 1,  _C5, _C5, None),
]

_OUTPUTS = [
    ("relu1_1", 64, _C1), ("relu1_2", 64, _C1),
    ("relu2_1", 32, _C2), ("relu2_2", 32, _C2),
    ("relu3_1", 16, _C3), ("relu3_2", 16, _C3), ("relu3_3", 16, _C3),
    ("max_3", 8, _C3),
    ("relu4_1", 8, _C4), ("relu4_2", 8, _C4), ("relu4_3", 8, _C4),
    ("relu5_1", 4, _C5), ("relu5_2", 2, _C5), ("relu5_3", 1, _C5),
]

_SCRATCH_ONLY = [("p1", 32, _C1), ("p2", 16, _C2), ("p3", 4, _C4),
                 ("p4", 2, _C5), ("p5", 1, _C5)]

_BUFFERS = [
    ("relu1_1", 64, _C1), ("relu1_2", 64, _C1), ("p1", 32, _C1),
    ("relu2_1", 32, _C2), ("relu2_2", 32, _C2), ("p2", 16, _C2),
    ("relu3_1", 16, _C3), ("relu3_2", 16, _C3), ("relu3_3", 16, _C3),
    ("max_3", 8, _C3),
    ("relu4_1", 8, _C4), ("relu4_2", 8, _C4), ("relu4_3", 8, _C4),
    ("p3", 4, _C4),
    ("relu5_1", 4, _C5), ("p4", 2, _C5), ("relu5_2", 2, _C5),
    ("p5", 1, _C5), ("relu5_3", 1, _C5),
]

_POOL_SIZES = [64, 32, 16, 8, 4, 2]


def _r(s):
    return (s + 2) * (s + 2)


def _pool_selector(s):
    wp, so = s + 2, s // 2
    wpo = so + 2
    k = (s - 1) * wp + s
    sel = np.zeros((k, wpo * wpo), np.float32)
    for i in range(so):
        for j in range(so):
            sel[(2 * i + 1) * wp + (2 * j + 1), (i + 1) * wpo + (j + 1)] = 1.0
    return sel.astype(jnp.bfloat16)


_SELS = {s: _pool_selector(s) for s in _POOL_SIZES}


def _interior_mask(s):
    wp, r = s + 2, _r(s)
    big = _BB * r
    m0, m1 = wp + 1, big - wp - 1
    col = np.arange(m0, m1)
    k = col % r
    h, w = k // wp, k % wp
    keep = (h >= 1) & (h <= s) & (w >= 1) & (w <= s)
    return keep[None, :].astype(np.float32)


_MASK_SIZES = [64, 32, 16, 8, 4, 2, 1]
_MASKS = {s: _interior_mask(s) for s in _MASK_SIZES}


def _conv3x3_relu(src, dst, w_ref, b_ref, sx, s, cin, cout, keep):
    wp, r = s + 2, _r(s)
    big = _BB * r
    m0, m1 = wp + 1, big - wp - 1
    mlen = m1 - m0
    k9 = 9 * cin

    offs = [(dy - 1) * wp + (dx - 1) for dy in range(3) for dx in range(3)]
    per = next((p for p in range(1, 9) if (p * cin) % 16 == 0), 9)
    t = 0
    while t < 9:
        g = offs[t:t + per]
        blk = jnp.concatenate(
            [src[:, m0 + o:m1 + o] for o in g], axis=0) if len(g) > 1 \
            else src[:, m0 + g[0]:m1 + g[0]]
        rows = len(g) * cin
        if rows % 16:
            blk = jnp.concatenate(
                [blk, jnp.zeros((16 - rows % 16, mlen), _DT)], axis=0)
            rows += 16 - rows % 16
        sx[t * cin:t * cin + rows, 0:mlen] = blk
        t += per

    acc = None
    for k0 in range(0, k9, 256):
        kc = min(256, k9 - k0)
        d = jnp.dot(w_ref[:, k0:k0 + kc], sx[k0:k0 + kc, 0:mlen],
                    preferred_element_type=jnp.float32)
        acc = d if acc is None else acc + d
    acc = jnp.maximum(acc + b_ref[...], 0.0) * keep[...]

    dst[:, 0:m0] = jnp.zeros((cout, m0), _DT)
    dst[:, m0:m1] = acc.astype(_DT)
    dst[:, m1:big] = jnp.zeros((cout, big - m1), _DT)


def _maxpool2x2(src, dst, sel_ref, s, c):
    wp, r = s + 2, _r(s)
    big = _BB * r
    ro = _r(s // 2)
    k = (s - 1) * wp + s

    v = src[...]
    a = jnp.maximum(v[:, 0:big - 1], v[:, 1:big])
    m2 = jnp.maximum(a[:, 0:big - 1 - wp], a[:, wp:big - 1])
    seg = jnp.concatenate([m2[:, e * r:e * r + k] for e in range(_BB)],
                          axis=0)
    o = jnp.dot(seg, sel_ref[...], preferred_element_type=jnp.float32)
    for e in range(_BB):
        dst[:, e * ro:(e + 1) * ro] = o[e * c:(e + 1) * c, :].astype(_DT)


def _net_kernel(*refs):
    pos = 0
    x_ref = refs[pos]
    pos += 1
    w, b = {}, {}
    for name, _, _ in _CONV_CFG:
        w[name], b[name] = refs[pos], refs[pos + 1]
        pos += 2
    sels = {}
    for s in _POOL_SIZES:
        sels[s] = refs[pos]
        pos += 1
    masks = {}
    for s in _MASK_SIZES:
        masks[s] = refs[pos]
        pos += 1
    outs = {}
    for name, _, _ in _OUTPUTS:
        outs[name] = refs[pos]
        pos += 1
    buf = {"x": x_ref}
    for name, _, _ in _BUFFERS:
        buf[name] = refs[pos]
        pos += 1
    sxa, sxb = refs[pos], refs[pos + 1]

    for cname, src, dst, s, cin, cout, pooled in _PLAN:
        _conv3x3_relu(buf[src], buf[dst], w[cname], b[cname],
                      sxa if s == 64 else sxb, s, cin, cout, masks[s])
        if pooled is not None:
            _maxpool2x2(buf[dst], buf[pooled], sels[s], s, cout)

    for name, s, c in _OUTPUTS:
        r = _r(s)
        src = buf[name]
        dst = outs[name]
        for e in range(_BB):
            dst[e] = src[:, e * r:(e + 1) * r]


def _build_call(batch):
    rx = _r(64)
    grid = (batch // _BB,)
    in_specs = [pl.BlockSpec((None, 3, _BB * rx), lambda n: (n, 0, 0))]
    for _, cin, cout in _CONV_CFG:
        in_specs.append(pl.BlockSpec((cout, 9 * cin), lambda n: (0, 0)))
        in_specs.append(pl.BlockSpec((cout, 1), lambda n: (0, 0)))
    for s in _POOL_SIZES:
        in_specs.append(pl.BlockSpec(_SELS[s].shape, lambda n: (0, 0)))
    for s in _MASK_SIZES:
        in_specs.append(pl.BlockSpec(_MASKS[s].shape, lambda n: (0, 0)))
    out_specs = tuple(pl.BlockSpec((_BB, c, _r(s)), lambda n: (n, 0, 0))
                      for _, s, c in _OUTPUTS)
    out_shape = tuple(jax.ShapeDtypeStruct((batch, c, _r(s)), _DT)
                      for _, s, c in _OUTPUTS)
    scratch = [pltpu.VMEM((c, _BB * _r(s)), _DT) for _, s, c in _BUFFERS]
    scratch.append(pltpu.VMEM((80, _BB * rx), _DT))
    scratch.append(pltpu.VMEM((288, _BB * _r(32)), _DT))
    return pl.pallas_call(
        _net_kernel,
        grid=grid,
        in_specs=in_specs,
        out_specs=out_specs,
        out_shape=out_shape,
        scratch_shapes=scratch,
        compiler_params=pltpu.CompilerParams(
            dimension_semantics=("parallel",)),
    )


def kernel(x, w_conv1_1, b_conv1_1, w_conv1_2, b_conv1_2,
           w_conv2_1, b_conv2_1, w_conv2_2, b_conv2_2,
           w_conv3_1, b_conv3_1, w_conv3_2, b_conv3_2, w_conv3_3, b_conv3_3,
           w_conv4_1, b_conv4_1, w_conv4_2, b_conv4_2, w_conv4_3, b_conv4_3,
           w_conv5_1, b_conv5_1):
    wmap = {
        "conv1_1": (w_conv1_1, b_conv1_1), "conv1_2": (w_conv1_2, b_conv1_2),
        "conv2_1": (w_conv2_1, b_conv2_1), "conv2_2": (w_conv2_2, b_conv2_2),
        "conv3_1": (w_conv3_1, b_conv3_1), "conv3_2": (w_conv3_2, b_conv3_2),
        "conv3_3": (w_conv3_3, b_conv3_3),
        "conv4_1": (w_conv4_1, b_conv4_1), "conv4_2": (w_conv4_2, b_conv4_2),
        "conv4_3": (w_conv4_3, b_conv4_3),
        "conv5_1": (w_conv5_1, b_conv5_1),
    }
    batch = x.shape[0]
    h = x.shape[2]

    xp = jnp.pad(x, ((0, 0), (0, 0), (1, 1), (1, 1)))
    xp = xp.reshape(batch // _BB, _BB, 3, _r(h)).astype(_DT)
    xp = xp.transpose(0, 2, 1, 3).reshape(batch // _BB, 3, _BB * _r(h))

    args = [xp]
    for name, cin, cout in _CONV_CFG:
        wm, bm = wmap[name]
        args.append(wm)
        args.append(bm)
    for s in _POOL_SIZES:
        args.append(_SELS[s])
    for s in _MASK_SIZES:
        args.append(_MASKS[s])

    outs = _build_call(batch)(*args)

    feats = {}
    for (name, s, c), arr in zip(_OUTPUTS, outs):
        a = arr.reshape(batch, c, s + 2, s + 2)[:, :, 1:-1, 1:-1]
        feats[name] = a.astype(jnp.float32)
    return feats
